# Initial kernel scaffold; baseline (speedup 1.0000x reference)
import jax
import jax.numpy as jnp
from jax.experimental import pallas as pl


def kernel(x, embed):
    raise NotImplementedError("write your pallas kernel here")


# speedup vs baseline: 1.3012x; 1.3012x over previous
import functools

import jax
import jax.numpy as jnp
from jax import lax
from jax.experimental import pallas as pl
from jax.experimental.pallas import tpu as pltpu
from jax.experimental.pallas import tpu_sc as plsc

_DIM = 64
_SZ = 2
_D = _SZ * _SZ * _DIM
_K = 8192
_N = 16384
_BLK = 256
_KC = 1024
_NKC = _K // _KC


def _vq_argmin_body(xf_ref, e_ref, ind_ref, dsum_ref):
    i = pl.program_id(0)
    x = xf_ref[...]
    x_sq = jnp.sum(x * x, axis=1, keepdims=True)
    xb = x.astype(jnp.bfloat16)
    best_val = jnp.full((_BLK, 1), jnp.inf, dtype=jnp.float32)
    best_idx = jnp.zeros((_BLK, 1), dtype=jnp.int32)
    for c in range(_NKC):
        e_c = e_ref[:, c * _KC:(c + 1) * _KC]
        m = lax.dot_general(xb, e_c, (((1,), (0,)), ((), ())),
                            preferred_element_type=jnp.float32)
        e_sq = jnp.sum(e_c * e_c, axis=0, keepdims=True)
        dist = x_sq - 2.0 * m + e_sq
        lmin = jnp.min(dist, axis=1, keepdims=True)
        cols = lax.broadcasted_iota(jnp.int32, (_BLK, _KC), 1)
        lidx = jnp.min(jnp.where(dist == lmin, cols, _K), axis=1,
                       keepdims=True) + c * _KC
        upd = lmin < best_val
        best_val = jnp.where(upd, lmin, best_val)
        best_idx = jnp.where(upd, lidx, best_idx)
    ind_ref[...] = best_idx

    @pl.when(i == 0)
    def _init():
        dsum_ref[0, 0] = 0.0

    dsum_ref[0, 0] += jnp.sum(best_val)


_vq_argmin = pl.pallas_call(
    _vq_argmin_body,
    grid=(_N // _BLK,),
    in_specs=[
        pl.BlockSpec((_BLK, _D), lambda i: (i, 0)),
        pl.BlockSpec((_D, _K), lambda i: (0, 0)),
    ],
    out_specs=[
        pl.BlockSpec((_BLK, 1), lambda i: (i, 0)),
        pl.BlockSpec(memory_space=pltpu.SMEM),
    ],
    out_shape=[
        jax.ShapeDtypeStruct((_N, 1), jnp.int32),
        jax.ShapeDtypeStruct((1, 1), jnp.float32),
    ],
    compiler_params=pltpu.CompilerParams(
        dimension_semantics=("arbitrary",)),
)


_NW = 32
_RPW = _N // _NW
_CH = 128
_NCHUNK = _RPW // _CH

@functools.cache
def _build_gather_codes():
    mesh = plsc.VectorSubcoreMesh(core_axis_name="c", subcore_axis_name="s")

    @functools.partial(
        pl.kernel,
        mesh=mesh,
        out_type=jax.ShapeDtypeStruct((_N, _D), jnp.float32),
        scratch_types=[
            pltpu.VMEM((_NCHUNK, _CH), jnp.int32),
            pltpu.VMEM((_CH, _D), jnp.float32),
            pltpu.VMEM((_CH, _D), jnp.float32),
            pltpu.SemaphoreType.DMA,
            pltpu.SemaphoreType.DMA,
        ],
    )
    def _gather_codes(table_hbm, idx_hbm, out_hbm,
                      idx_v, buf0, buf1, sem0, sem1):
        wid = lax.axis_index("s") * 2 + lax.axis_index("c")
        base = wid * _RPW
        pltpu.sync_copy(idx_hbm.at[pl.ds(wid * _NCHUNK, _NCHUNK)], idx_v)
        bufs = (buf0, buf1)
        sems = (sem0, sem1)
        prev = pltpu.async_copy(table_hbm.at[idx_v.at[0]], buf0, sem0)
        for c in range(1, _NCHUNK):
            nxt = pltpu.async_copy(table_hbm.at[idx_v.at[c]], bufs[c % 2],
                                   sems[c % 2])
            prev.wait()
            pltpu.sync_copy(bufs[(c - 1) % 2],
                            out_hbm.at[pl.ds(base + (c - 1) * _CH, _CH)])
            prev = nxt
        prev.wait()
        pltpu.sync_copy(bufs[(_NCHUNK - 1) % 2],
                        out_hbm.at[pl.ds(base + (_NCHUNK - 1) * _CH, _CH)])

    return _gather_codes


def kernel(x, embed):
    bs, hH, _, C = x.shape
    rH = hH // _SZ
    xr = x.reshape(bs, rH, _SZ, rH, _SZ, C).transpose(0, 1, 3, 2, 4, 5)
    flat = xr.reshape(_N, _D)
    e = embed.reshape(_D, _K)

    ind2, dsum = _vq_argmin(flat, e)
    ind = ind2.reshape(_N)

    codebook = jnp.transpose(embed, (3, 0, 1, 2)).reshape(_K, _D)
    quant = _build_gather_codes()(codebook, ind.reshape(_N // _CH, _CH))

    q = quant.reshape(bs, rH, rH, _SZ, _SZ, C)
    q = q.transpose(0, 1, 3, 2, 4, 5).reshape(bs, hH, hH, C)

    diff = dsum[0, 0] * jnp.float32(1.0 / (_N * _D))
    embed_ind = ind.reshape(bs, rH, rH)
    avg = jnp.float32(1.0 / _K)
    perplexity = jnp.exp(-avg * jnp.log(avg + 1e-10))
    return (q, diff, embed_ind, perplexity)
